# baseline (speedup 1.0000x reference)
import functools

import jax
import jax.numpy as jnp
from jax import lax
from jax.experimental import pallas as pl
from jax.experimental.pallas import tpu as pltpu
from jax.experimental.pallas import tpu_sc as plsc

NUM_FRAMES = 32
T = 300
IDX = [(i * (T - 1)) // (NUM_FRAMES - 1) for i in range(NUM_FRAMES)]

SC_FRAMES = 8
TC_FRAMES = NUM_FRAMES - SC_FRAMES

CROWS = 56
CPP = 224 // CROWS
CPF = 3 * CPP

_info = plsc.get_sparse_core_info()
_NC, _NS = _info.num_cores, _info.num_subcores
NW = _NC * _NS
CPT = SC_FRAMES * CPF // NW
NBUF = min(CPT, 8)

NSEM = 8


def _sc_gather_kernel(frames_hbm, out_hbm, *scratch):
    bufs = scratch[:NBUF]
    sins = scratch[NBUF:2 * NBUF]
    souts = scratch[2 * NBUF:]

    wid = lax.axis_index("s") * _NC + lax.axis_index("c")

    def coords(c):
        g = wid * CPT + c
        f = g // CPF
        ch, r = (g % CPF) // CPP, ((g % CPF) % CPP) * CROWS
        src = (f + TC_FRAMES) * (T - 1) // (NUM_FRAMES - 1)
        return f, src, ch, r

    def in_copy(c):
        f, src, ch, r = coords(c)
        return pltpu.make_async_copy(
            frames_hbm.at[src, ch, pl.ds(r, CROWS)], bufs[c % NBUF], sins[c % NBUF]
        )

    def out_copy(c):
        f, src, ch, r = coords(c)
        return pltpu.make_async_copy(
            bufs[c % NBUF], out_hbm.at[f, ch, pl.ds(r, CROWS)], souts[c % NBUF]
        )

    for c in range(min(NBUF, CPT)):
        in_copy(c).start()
    for c in range(CPT):
        in_copy(c).wait()
        out_copy(c).start()
        if c + NBUF < CPT:
            out_copy(c).wait()
            in_copy(c + NBUF).start()
    for c in range(max(0, CPT - NBUF), CPT):
        out_copy(c).wait()


def _tc_copy_kernel(frames_hbm, sc_hbm, out_hbm, *sems):
    copies = []
    for j in range(TC_FRAMES):
        copies.append(
            pltpu.make_async_copy(
                frames_hbm.at[IDX[j]], out_hbm.at[j], sems[j % NSEM]
            )
        )
    for j in range(SC_FRAMES):
        copies.append(
            pltpu.make_async_copy(
                sc_hbm.at[j], out_hbm.at[TC_FRAMES + j], sems[(TC_FRAMES + j) % NSEM]
            )
        )
    for cp in copies:
        cp.start()
    for cp in copies:
        cp.wait()


@jax.jit
def kernel(frames):
    mesh = plsc.VectorSubcoreMesh(core_axis_name="c", subcore_axis_name="s")
    sc_out = pl.kernel(
        _sc_gather_kernel,
        out_type=jax.ShapeDtypeStruct((SC_FRAMES, 3, 224, 224), jnp.float32),
        mesh=mesh,
        scratch_types=(
            [pltpu.VMEM((CROWS, 224), jnp.float32)] * NBUF
            + [pltpu.SemaphoreType.DMA] * (2 * NBUF)
        ),
    )(frames)

    return pl.pallas_call(
        _tc_copy_kernel,
        in_specs=[
            pl.BlockSpec(memory_space=pl.ANY),
            pl.BlockSpec(memory_space=pl.ANY),
        ],
        out_specs=pl.BlockSpec(memory_space=pl.ANY),
        out_shape=jax.ShapeDtypeStruct((NUM_FRAMES, 3, 224, 224), jnp.float32),
        scratch_shapes=[pltpu.SemaphoreType.DMA] * NSEM,
    )(frames, sc_out)


# speedup vs baseline: 13.9715x; 13.9715x over previous
import jax
import jax.numpy as jnp
from jax import lax
from jax.experimental import pallas as pl
from jax.experimental.pallas import tpu as pltpu
from jax.experimental.pallas import tpu_sc as plsc

NUM_FRAMES = 32
T = 300
IDX = [(i * (T - 1)) // (NUM_FRAMES - 1) for i in range(NUM_FRAMES)]

SC_FRAMES = 8
TC_FRAMES = NUM_FRAMES - SC_FRAMES

CROWS = 56
CPP = 224 // CROWS
CPF = 3 * CPP

_info = plsc.get_sparse_core_info()
_NC, _NS = _info.num_cores, _info.num_subcores
NW = _NC * _NS
CPT = SC_FRAMES * CPF // NW
NBUF = min(CPT, 8)

TC_NBUF = 8


def _sc_gather_kernel(frames_hbm, out_hbm, *scratch):
    bufs = scratch[:NBUF]
    sins = scratch[NBUF:2 * NBUF]
    souts = scratch[2 * NBUF:]

    wid = lax.axis_index("s") * _NC + lax.axis_index("c")

    def coords(c):
        g = wid * CPT + c
        f = g // CPF
        ch, r = (g % CPF) // CPP, ((g % CPF) % CPP) * CROWS
        src = (f + TC_FRAMES) * (T - 1) // (NUM_FRAMES - 1)
        return f, src, ch, r

    def in_copy(c):
        f, src, ch, r = coords(c)
        return pltpu.make_async_copy(
            frames_hbm.at[src, ch, pl.ds(r, CROWS)], bufs[c % NBUF], sins[c % NBUF]
        )

    def out_copy(c):
        f, src, ch, r = coords(c)
        return pltpu.make_async_copy(
            bufs[c % NBUF], out_hbm.at[f, ch, pl.ds(r, CROWS)], souts[c % NBUF]
        )

    for c in range(min(NBUF, CPT)):
        in_copy(c).start()
    for c in range(CPT):
        in_copy(c).wait()
        out_copy(c).start()
        if c + NBUF < CPT:
            out_copy(c).wait()
            in_copy(c + NBUF).start()
    for c in range(max(0, CPT - NBUF), CPT):
        out_copy(c).wait()


def _tc_copy_kernel(frames_hbm, sc_hbm, out_hbm, *scratch):
    bufs = scratch[:TC_NBUF]
    sins = scratch[TC_NBUF:2 * TC_NBUF]
    souts = scratch[2 * TC_NBUF:]

    def src_ref(j):
        if j < TC_FRAMES:
            return frames_hbm.at[IDX[j]]
        return sc_hbm.at[j - TC_FRAMES]

    def in_copy(j):
        return pltpu.make_async_copy(src_ref(j), bufs[j % TC_NBUF], sins[j % TC_NBUF])

    def out_copy(j):
        return pltpu.make_async_copy(bufs[j % TC_NBUF], out_hbm.at[j], souts[j % TC_NBUF])

    for j in range(min(TC_NBUF, NUM_FRAMES)):
        in_copy(j).start()
    for j in range(NUM_FRAMES):
        in_copy(j).wait()
        out_copy(j).start()
        if j + TC_NBUF < NUM_FRAMES:
            out_copy(j).wait()
            in_copy(j + TC_NBUF).start()
    for j in range(max(0, NUM_FRAMES - TC_NBUF), NUM_FRAMES):
        out_copy(j).wait()


@jax.jit
def kernel(frames):
    mesh = plsc.VectorSubcoreMesh(core_axis_name="c", subcore_axis_name="s")
    sc_out = pl.kernel(
        _sc_gather_kernel,
        out_type=jax.ShapeDtypeStruct((SC_FRAMES, 3, 224, 224), jnp.float32),
        mesh=mesh,
        scratch_types=(
            [pltpu.VMEM((CROWS, 224), jnp.float32)] * NBUF
            + [pltpu.SemaphoreType.DMA] * (2 * NBUF)
        ),
    )(frames)

    return pl.pallas_call(
        _tc_copy_kernel,
        in_specs=[
            pl.BlockSpec(memory_space=pl.ANY),
            pl.BlockSpec(memory_space=pl.ANY),
        ],
        out_specs=pl.BlockSpec(memory_space=pl.ANY),
        out_shape=jax.ShapeDtypeStruct((NUM_FRAMES, 3, 224, 224), jnp.float32),
        scratch_shapes=(
            [pltpu.VMEM((3, 224, 224), jnp.float32)] * TC_NBUF
            + [pltpu.SemaphoreType.DMA] * (2 * TC_NBUF)
        ),
    )(frames, sc_out)


# speedup vs baseline: 18.1046x; 1.2958x over previous
import jax
import jax.numpy as jnp
from jax import lax
from jax.experimental import pallas as pl
from jax.experimental.pallas import tpu as pltpu
from jax.experimental.pallas import tpu_sc as plsc

NUM_FRAMES = 32
T = 300
IDX = [(i * (T - 1)) // (NUM_FRAMES - 1) for i in range(NUM_FRAMES)]

SC_FRAMES = 8
TC_FRAMES = NUM_FRAMES - SC_FRAMES

CROWS = 56
CPP = 224 // CROWS
CPF = 3 * CPP

_info = plsc.get_sparse_core_info()
_NC, _NS = _info.num_cores, _info.num_subcores
NW = _NC * _NS
CPT = SC_FRAMES * CPF // NW
NBUF = min(CPT, 8)

TC_NBUF = 32


def _sc_gather_kernel(frames_hbm, out_hbm, *scratch):
    bufs = scratch[:NBUF]
    sins = scratch[NBUF:2 * NBUF]
    souts = scratch[2 * NBUF:]

    wid = lax.axis_index("s") * _NC + lax.axis_index("c")

    def coords(c):
        g = wid * CPT + c
        f = g // CPF
        ch, r = (g % CPF) // CPP, ((g % CPF) % CPP) * CROWS
        src = (f + TC_FRAMES) * (T - 1) // (NUM_FRAMES - 1)
        return f, src, ch, r

    def in_copy(c):
        f, src, ch, r = coords(c)
        return pltpu.make_async_copy(
            frames_hbm.at[src, ch, pl.ds(r, CROWS)], bufs[c % NBUF], sins[c % NBUF]
        )

    def out_copy(c):
        f, src, ch, r = coords(c)
        return pltpu.make_async_copy(
            bufs[c % NBUF], out_hbm.at[f, ch, pl.ds(r, CROWS)], souts[c % NBUF]
        )

    for c in range(min(NBUF, CPT)):
        in_copy(c).start()
    for c in range(CPT):
        in_copy(c).wait()
        out_copy(c).start()
        if c + NBUF < CPT:
            out_copy(c).wait()
            in_copy(c + NBUF).start()
    for c in range(max(0, CPT - NBUF), CPT):
        out_copy(c).wait()


def _tc_copy_kernel(frames_hbm, sc_hbm, out_hbm, *scratch):
    bufs = scratch[:TC_NBUF]
    sins = scratch[TC_NBUF:2 * TC_NBUF]
    souts = scratch[2 * TC_NBUF:]

    def src_ref(j):
        if j < TC_FRAMES:
            return frames_hbm.at[IDX[j]]
        return sc_hbm.at[j - TC_FRAMES]

    def in_copy(j):
        return pltpu.make_async_copy(src_ref(j), bufs[j % TC_NBUF], sins[j % TC_NBUF])

    def out_copy(j):
        return pltpu.make_async_copy(bufs[j % TC_NBUF], out_hbm.at[j], souts[j % TC_NBUF])

    for j in range(min(TC_NBUF, NUM_FRAMES)):
        in_copy(j).start()
    for j in range(NUM_FRAMES):
        in_copy(j).wait()
        out_copy(j).start()
        if j + TC_NBUF < NUM_FRAMES:
            out_copy(j).wait()
            in_copy(j + TC_NBUF).start()
    for j in range(max(0, NUM_FRAMES - TC_NBUF), NUM_FRAMES):
        out_copy(j).wait()


@jax.jit
def kernel(frames):
    mesh = plsc.VectorSubcoreMesh(core_axis_name="c", subcore_axis_name="s")
    sc_out = pl.kernel(
        _sc_gather_kernel,
        out_type=jax.ShapeDtypeStruct((SC_FRAMES, 3, 224, 224), jnp.float32),
        mesh=mesh,
        scratch_types=(
            [pltpu.VMEM((CROWS, 224), jnp.float32)] * NBUF
            + [pltpu.SemaphoreType.DMA] * (2 * NBUF)
        ),
    )(frames)

    return pl.pallas_call(
        _tc_copy_kernel,
        in_specs=[
            pl.BlockSpec(memory_space=pl.ANY),
            pl.BlockSpec(memory_space=pl.ANY),
        ],
        out_specs=pl.BlockSpec(memory_space=pl.ANY),
        out_shape=jax.ShapeDtypeStruct((NUM_FRAMES, 3, 224, 224), jnp.float32),
        scratch_shapes=(
            [pltpu.VMEM((3, 224, 224), jnp.float32)] * TC_NBUF
            + [pltpu.SemaphoreType.DMA] * (2 * TC_NBUF)
        ),
    )(frames, sc_out)


# speedup vs baseline: 19.4279x; 1.0731x over previous
import jax
import jax.numpy as jnp
from jax import lax
from jax.experimental import pallas as pl
from jax.experimental.pallas import tpu as pltpu
from jax.experimental.pallas import tpu_sc as plsc

NUM_FRAMES = 32
T = 300
IDX = [(i * (T - 1)) // (NUM_FRAMES - 1) for i in range(NUM_FRAMES)]

SC_FRAMES = 8
TC_FRAMES = NUM_FRAMES - SC_FRAMES

CROWS = 56
CPP = 224 // CROWS
CPF = 3 * CPP

_info = plsc.get_sparse_core_info()
_NC, _NS = _info.num_cores, _info.num_subcores
NW = _NC * _NS
CPT = SC_FRAMES * CPF // NW
NBUF = min(CPT, 8)

TC_NBUF = 32


def _sc_gather_kernel(frames_hbm, out_hbm, *scratch):
    bufs = scratch[:NBUF]
    sins = scratch[NBUF:2 * NBUF]
    souts = scratch[2 * NBUF:]

    wid = lax.axis_index("s") * _NC + lax.axis_index("c")

    def coords(c):
        g = wid * CPT + c
        f = g // CPF
        ch, r = (g % CPF) // CPP, ((g % CPF) % CPP) * CROWS
        src = (f + TC_FRAMES) * (T - 1) // (NUM_FRAMES - 1)
        return f, src, ch, r

    def in_copy(c):
        f, src, ch, r = coords(c)
        return pltpu.make_async_copy(
            frames_hbm.at[src, ch, pl.ds(r, CROWS)], bufs[c % NBUF], sins[c % NBUF]
        )

    def out_copy(c):
        f, src, ch, r = coords(c)
        return pltpu.make_async_copy(
            bufs[c % NBUF], out_hbm.at[f, ch, pl.ds(r, CROWS)], souts[c % NBUF]
        )

    for c in range(min(NBUF, CPT)):
        in_copy(c).start()
    for c in range(CPT):
        in_copy(c).wait()
        out_copy(c).start()
        if c + NBUF < CPT:
            out_copy(c).wait()
            in_copy(c + NBUF).start()
    for c in range(max(0, CPT - NBUF), CPT):
        out_copy(c).wait()


def _tc_gather_kernel(frames_hbm, out_hbm, *scratch):
    bufs = scratch[:TC_FRAMES]
    sins = scratch[TC_FRAMES:2 * TC_FRAMES]
    souts = scratch[2 * TC_FRAMES:]

    def in_copy(j):
        return pltpu.make_async_copy(frames_hbm.at[IDX[j]], bufs[j], sins[j])

    def out_copy(j):
        return pltpu.make_async_copy(bufs[j], out_hbm.at[j], souts[j])

    for j in range(TC_FRAMES):
        in_copy(j).start()
    for j in range(TC_FRAMES):
        in_copy(j).wait()
        out_copy(j).start()
    for j in range(TC_FRAMES):
        out_copy(j).wait()


def _tc_stitch_kernel(base_hbm, sc_hbm, out_hbm, *scratch):
    bufs = scratch[:SC_FRAMES]
    sins = scratch[SC_FRAMES:2 * SC_FRAMES]
    souts = scratch[2 * SC_FRAMES:]
    for j in range(SC_FRAMES):
        pltpu.make_async_copy(sc_hbm.at[j], bufs[j], sins[j]).start()
    for j in range(SC_FRAMES):
        pltpu.make_async_copy(sc_hbm.at[j], bufs[j], sins[j]).wait()
        pltpu.make_async_copy(bufs[j], out_hbm.at[TC_FRAMES + j], souts[j]).start()
    for j in range(SC_FRAMES):
        pltpu.make_async_copy(bufs[j], out_hbm.at[TC_FRAMES + j], souts[j]).wait()


@jax.jit
def kernel(frames):
    mesh = plsc.VectorSubcoreMesh(core_axis_name="c", subcore_axis_name="s")
    sc_out = pl.kernel(
        _sc_gather_kernel,
        out_type=jax.ShapeDtypeStruct((SC_FRAMES, 3, 224, 224), jnp.float32),
        mesh=mesh,
        scratch_types=(
            [pltpu.VMEM((CROWS, 224), jnp.float32)] * NBUF
            + [pltpu.SemaphoreType.DMA] * (2 * NBUF)
        ),
    )(frames)

    tc_out = pl.pallas_call(
        _tc_gather_kernel,
        in_specs=[pl.BlockSpec(memory_space=pl.ANY)],
        out_specs=pl.BlockSpec(memory_space=pl.ANY),
        out_shape=jax.ShapeDtypeStruct((NUM_FRAMES, 3, 224, 224), jnp.float32),
        scratch_shapes=(
            [pltpu.VMEM((3, 224, 224), jnp.float32)] * TC_FRAMES
            + [pltpu.SemaphoreType.DMA] * (2 * TC_FRAMES)
        ),
    )(frames)

    return pl.pallas_call(
        _tc_stitch_kernel,
        in_specs=[
            pl.BlockSpec(memory_space=pl.ANY),
            pl.BlockSpec(memory_space=pl.ANY),
        ],
        out_specs=pl.BlockSpec(memory_space=pl.ANY),
        out_shape=jax.ShapeDtypeStruct((NUM_FRAMES, 3, 224, 224), jnp.float32),
        input_output_aliases={0: 0},
        scratch_shapes=(
            [pltpu.VMEM((3, 224, 224), jnp.float32)] * SC_FRAMES
            + [pltpu.SemaphoreType.DMA] * (2 * SC_FRAMES)
        ),
    )(tc_out, sc_out)


# speedup vs baseline: 19.9960x; 1.0292x over previous
import jax
import jax.numpy as jnp
from jax import lax
from jax.experimental import pallas as pl
from jax.experimental.pallas import tpu as pltpu
from jax.experimental.pallas import tpu_sc as plsc

NUM_FRAMES = 32
T = 300

CROWS = 56
CPP = 224 // CROWS
CPF = 3 * CPP
NBUF = 8

_info = plsc.get_sparse_core_info()
_NC, _NS = _info.num_cores, _info.num_subcores


def _clip_sampler_kernel(frames_hbm, out_hbm, *scratch):
    bufs = scratch[:NBUF]
    sins = scratch[NBUF:2 * NBUF]
    souts = scratch[2 * NBUF:]

    wid = lax.axis_index("s") * _NC + lax.axis_index("c")
    src = (wid * (T - 1)) // (NUM_FRAMES - 1)

    def in_copy(c):
        ch, r = c // CPP, (c % CPP) * CROWS
        return pltpu.make_async_copy(
            frames_hbm.at[src, ch, pl.ds(r, CROWS)], bufs[c % NBUF], sins[c % NBUF]
        )

    def out_copy(c):
        ch, r = c // CPP, (c % CPP) * CROWS
        return pltpu.make_async_copy(
            bufs[c % NBUF], out_hbm.at[wid, ch, pl.ds(r, CROWS)], souts[c % NBUF]
        )

    for c in range(min(NBUF, CPF)):
        in_copy(c).start()
    for c in range(CPF):
        in_copy(c).wait()
        out_copy(c).start()
        if c + NBUF < CPF:
            out_copy(c).wait()
            in_copy(c + NBUF).start()
    for c in range(max(0, CPF - NBUF), CPF):
        out_copy(c).wait()


@jax.jit
def kernel(frames):
    mesh = plsc.VectorSubcoreMesh(core_axis_name="c", subcore_axis_name="s")
    return pl.kernel(
        _clip_sampler_kernel,
        out_type=jax.ShapeDtypeStruct((NUM_FRAMES, 3, 224, 224), jnp.float32),
        mesh=mesh,
        scratch_types=(
            [pltpu.VMEM((CROWS, 224), jnp.float32)] * NBUF
            + [pltpu.SemaphoreType.DMA] * (2 * NBUF)
        ),
    )(frames)


# speedup vs baseline: 20.3789x; 1.0192x over previous
import jax
import jax.numpy as jnp
from jax import lax
from jax.experimental import pallas as pl
from jax.experimental.pallas import tpu as pltpu
from jax.experimental.pallas import tpu_sc as plsc

NUM_FRAMES = 32
T = 300

CROWS = 112
CPP = 224 // CROWS
CPF = 3 * CPP
NBUF = 4

_info = plsc.get_sparse_core_info()
_NC, _NS = _info.num_cores, _info.num_subcores


def _clip_sampler_kernel(frames_hbm, out_hbm, *scratch):
    bufs = scratch[:NBUF]
    sins = scratch[NBUF:2 * NBUF]
    souts = scratch[2 * NBUF:]

    wid = lax.axis_index("s") * _NC + lax.axis_index("c")
    src = (wid * (T - 1)) // (NUM_FRAMES - 1)

    def in_copy(c):
        ch, r = c // CPP, (c % CPP) * CROWS
        return pltpu.make_async_copy(
            frames_hbm.at[src, ch, pl.ds(r, CROWS)], bufs[c % NBUF], sins[c % NBUF]
        )

    def out_copy(c):
        ch, r = c // CPP, (c % CPP) * CROWS
        return pltpu.make_async_copy(
            bufs[c % NBUF], out_hbm.at[wid, ch, pl.ds(r, CROWS)], souts[c % NBUF]
        )

    for c in range(min(NBUF, CPF)):
        in_copy(c).start()
    for c in range(CPF):
        in_copy(c).wait()
        out_copy(c).start()
        if c + NBUF < CPF:
            out_copy(c).wait()
            in_copy(c + NBUF).start()
    for c in range(max(0, CPF - NBUF), CPF):
        out_copy(c).wait()


@jax.jit
def kernel(frames):
    mesh = plsc.VectorSubcoreMesh(core_axis_name="c", subcore_axis_name="s")
    return pl.kernel(
        _clip_sampler_kernel,
        out_type=jax.ShapeDtypeStruct((NUM_FRAMES, 3, 224, 224), jnp.float32),
        mesh=mesh,
        scratch_types=(
            [pltpu.VMEM((CROWS, 224), jnp.float32)] * NBUF
            + [pltpu.SemaphoreType.DMA] * (2 * NBUF)
        ),
    )(frames)
